# Initial kernel scaffold; baseline (speedup 1.0000x reference)
import jax
import jax.numpy as jnp
from jax.experimental import pallas as pl


def kernel(inputs, atom_table, num_table):
    raise NotImplementedError("write your pallas kernel here")


# speedup vs baseline: 4.2410x; 4.2410x over previous
import functools

import jax
import jax.numpy as jnp
from jax import lax
from jax.experimental import pallas as pl
from jax.experimental.pallas import tpu as pltpu
from jax.experimental.pallas import tpu_sc as plsc

B, L = 1024, 200
N = B * L
IN_W = 41
OUT_W = 160
DIM = 128
NDIM = 16

_INFO = plsc.get_sparse_core_info()
NC, NS, LANES = _INFO.num_cores, _INFO.num_subcores, _INFO.num_lanes
NW = NC * NS
TPW = N // NW
C = 256
NCHUNK = TPW // C
GPC = C // LANES


def _body(in_hbm, atom_hbm, num_hbm, out_hbm, atom_v, num_v, in_v, out_v):
    wid = lax.axis_index("s") * NC + lax.axis_index("c")

    pltpu.sync_copy(atom_hbm, atom_v)
    pltpu.sync_copy(num_hbm, num_v)

    iota = lax.iota(jnp.int32, LANES)

    def chunk_body(ci, _):
        tb = wid * TPW + ci * C

        pltpu.sync_copy(in_hbm.at[pl.ds(tb * IN_W, C * IN_W)], in_v)

        def group_body(g, _):
            tv = iota + g * LANES
            ibase = tv * IN_W
            obase = tv * OUT_W

            names = plsc.load_gather(in_v, [ibase]).astype(jnp.int32)
            abase = names * DIM
            nbases = []
            for j in range(8):
                nid = plsc.load_gather(in_v, [ibase + (33 + j)])
                nbases.append(nid.astype(jnp.int32) * NDIM)

            for c in range(DIM):
                j, m = c // NDIM, c % NDIM
                av = plsc.load_gather(atom_v, [abase + c])
                nv = plsc.load_gather(num_v, [nbases[j] + m])
                plsc.store_scatter(out_v, [obase + c], av + nv)

            for c in range(DIM, OUT_W):
                s = (c - 124) if c < 157 else (c - 156)
                v = plsc.load_gather(in_v, [ibase + s])
                plsc.store_scatter(out_v, [obase + c], v)
            return _

        lax.fori_loop(0, GPC, group_body, None)

        pltpu.sync_copy(out_v, out_hbm.at[pl.ds(tb * OUT_W, C * OUT_W)])
        return _

    lax.fori_loop(0, NCHUNK, chunk_body, None)


def kernel(inputs, atom_table, num_table):
    mesh = plsc.VectorSubcoreMesh(core_axis_name="c", subcore_axis_name="s")
    run = functools.partial(
        pl.kernel,
        mesh=mesh,
        compiler_params=pltpu.CompilerParams(needs_layout_passes=False),
        out_type=jax.ShapeDtypeStruct((N * OUT_W,), jnp.float32),
        scratch_types=[
            pltpu.VMEM((100 * DIM,), jnp.float32),
            pltpu.VMEM((500 * NDIM,), jnp.float32),
            pltpu.VMEM((C * IN_W,), jnp.float32),
            pltpu.VMEM((C * OUT_W,), jnp.float32),
        ],
    )(_body)
    out = run(
        inputs.reshape(-1),
        atom_table.reshape(-1),
        num_table.reshape(-1),
    )
    return out.reshape(B, L, OUT_W)


# speedup vs baseline: 6.1811x; 1.4575x over previous
import functools

import jax
import jax.numpy as jnp
from jax import lax
from jax.experimental import pallas as pl
from jax.experimental.pallas import tpu as pltpu
from jax.experimental.pallas import tpu_sc as plsc

B, L = 1024, 200
N = B * L
IN_W = 41
OUT_W = 160
DIM = 128
AP = 129
NDIM = 16
NP = 17

_INFO = plsc.get_sparse_core_info()
NC, NS, LANES = _INFO.num_cores, _INFO.num_subcores, _INFO.num_lanes
NW = NC * NS
TPW = N // NW
C = 256
NCHUNK = TPW // C
GPC = C // LANES


def _body(in_hbm, atom_hbm, num_hbm, out_hbm, atom_v, num_v, in_v, out_v):
    wid = lax.axis_index("s") * NC + lax.axis_index("c")

    pltpu.sync_copy(atom_hbm, atom_v)
    pltpu.sync_copy(num_hbm, num_v)

    iota = lax.iota(jnp.int32, LANES)

    def chunk_body(ci, _):
        tb = wid * TPW + ci * C

        pltpu.sync_copy(in_hbm.at[pl.ds(tb * IN_W, C * IN_W)], in_v)

        def group_body(g, _):
            tv = iota + g * LANES
            ibase = tv * IN_W
            obase = tv * OUT_W

            names = plsc.load_gather(in_v, [ibase]).astype(jnp.int32)
            abase = names * AP
            nbases = []
            for j in range(8):
                nid = plsc.load_gather(in_v, [ibase + (33 + j)])
                nbases.append(nid.astype(jnp.int32) * NP)

            for c in range(DIM):
                j, m = c // NDIM, c % NDIM
                av = plsc.load_gather(atom_v, [abase + c])
                nv = plsc.load_gather(num_v, [nbases[j] + m])
                plsc.store_scatter(out_v, [obase + c], av + nv)

            for c in range(DIM, OUT_W):
                s = (c - 124) if c < 157 else (c - 156)
                v = plsc.load_gather(in_v, [ibase + s])
                plsc.store_scatter(out_v, [obase + c], v)
            return _

        lax.fori_loop(0, GPC, group_body, None)

        pltpu.sync_copy(out_v, out_hbm.at[pl.ds(tb * OUT_W, C * OUT_W)])
        return _

    lax.fori_loop(0, NCHUNK, chunk_body, None)


def kernel(inputs, atom_table, num_table):
    mesh = plsc.VectorSubcoreMesh(core_axis_name="c", subcore_axis_name="s")
    run = functools.partial(
        pl.kernel,
        mesh=mesh,
        compiler_params=pltpu.CompilerParams(needs_layout_passes=False),
        out_type=jax.ShapeDtypeStruct((N * OUT_W,), jnp.float32),
        scratch_types=[
            pltpu.VMEM((100 * AP,), jnp.float32),
            pltpu.VMEM((500 * NP,), jnp.float32),
            pltpu.VMEM((C * IN_W,), jnp.float32),
            pltpu.VMEM((C * OUT_W,), jnp.float32),
        ],
    )(_body)
    atom_p = jnp.pad(atom_table, ((0, 0), (0, AP - DIM))).reshape(-1)
    num_p = jnp.pad(num_table, ((0, 0), (0, NP - NDIM))).reshape(-1)
    out = run(inputs.reshape(-1), atom_p, num_p)
    return out.reshape(B, L, OUT_W)


# speedup vs baseline: 8.6422x; 1.3982x over previous
import functools

import jax
import jax.numpy as jnp
from jax import lax
from jax.experimental import pallas as pl
from jax.experimental.pallas import tpu as pltpu
from jax.experimental.pallas import tpu_sc as plsc

B, L = 1024, 200
N = B * L
IN_W = 41
OUT_W = 160
DIM = 128
AP = 129
NDIM = 16
NP = 17

_INFO = plsc.get_sparse_core_info()
NC, NS, LANES = _INFO.num_cores, _INFO.num_subcores, _INFO.num_lanes
NW = NC * NS
TPW = N // NW
C = 256
NCHUNK = TPW // C
GPC = C // LANES


def _body(in_hbm, atom_hbm, num_hbm, out_hbm, atom_v, num_v, in_v, out_v):
    wid = lax.axis_index("s") * NC + lax.axis_index("c")

    pltpu.sync_copy(atom_hbm, atom_v)
    pltpu.sync_copy(num_hbm, num_v)

    iota = lax.iota(jnp.int32, LANES)
    mvecs = [(iota + c) & (LANES - 1) for c in range(LANES)]

    def chunk_body(ci, _):
        tb = wid * TPW + ci * C

        pltpu.sync_copy(in_hbm.at[pl.ds(tb * IN_W, C * IN_W)], in_v)

        def group_body(g, _):
            tv = iota + g * LANES
            ibase = tv * IN_W
            obase = tv * OUT_W

            names = plsc.load_gather(in_v, [ibase]).astype(jnp.int32)
            abase = names * AP
            nbases = []
            for j in range(8):
                nid = plsc.load_gather(in_v, [ibase + (33 + j)])
                nbases.append(nid.astype(jnp.int32) * NP)

            for blk in range(DIM // NDIM):
                ab = abase + blk * NDIM
                ob = obase + blk * NDIM
                nb = nbases[blk]
                for c in range(NDIM):
                    mv = mvecs[c]
                    av = plsc.load_gather(atom_v, [ab + mv])
                    nv = plsc.load_gather(num_v, [nb + mv])
                    plsc.store_scatter(out_v, [ob + mv], av + nv)

            ob = obase + DIM
            for c in range(NDIM):
                mv = mvecs[c]
                v = plsc.load_gather(in_v, [ibase + 4 + mv])
                plsc.store_scatter(out_v, [ob + mv], v)
            ob = obase + DIM + NDIM
            for c in range(NDIM):
                mv = mvecs[c]
                sv = jnp.where(mv <= 12, mv + 20, mv - 12)
                v = plsc.load_gather(in_v, [ibase + sv])
                plsc.store_scatter(out_v, [ob + mv], v)
            return _

        lax.fori_loop(0, GPC, group_body, None)

        pltpu.sync_copy(out_v, out_hbm.at[pl.ds(tb * OUT_W, C * OUT_W)])
        return _

    lax.fori_loop(0, NCHUNK, chunk_body, None)


def kernel(inputs, atom_table, num_table):
    mesh = plsc.VectorSubcoreMesh(core_axis_name="c", subcore_axis_name="s")
    run = functools.partial(
        pl.kernel,
        mesh=mesh,
        compiler_params=pltpu.CompilerParams(needs_layout_passes=False),
        out_type=jax.ShapeDtypeStruct((N * OUT_W,), jnp.float32),
        scratch_types=[
            pltpu.VMEM((100 * AP,), jnp.float32),
            pltpu.VMEM((500 * NP,), jnp.float32),
            pltpu.VMEM((C * IN_W,), jnp.float32),
            pltpu.VMEM((C * OUT_W,), jnp.float32),
        ],
    )(_body)
    atom_p = jnp.pad(atom_table, ((0, 0), (0, AP - DIM))).reshape(-1)
    num_p = jnp.pad(num_table, ((0, 0), (0, NP - NDIM))).reshape(-1)
    out = run(inputs.reshape(-1), atom_p, num_p)
    return out.reshape(B, L, OUT_W)


# speedup vs baseline: 11.3214x; 1.3100x over previous
import functools

import jax
import jax.numpy as jnp
from jax import lax
from jax.experimental import pallas as pl
from jax.experimental.pallas import tpu as pltpu
from jax.experimental.pallas import tpu_sc as plsc

B, L = 1024, 200
N = B * L
IN_W = 41
OUT_W = 160
DIM = 128
AP = 129
NDIM = 16
NP = 17

_INFO = plsc.get_sparse_core_info()
NC, NS, LANES = _INFO.num_cores, _INFO.num_subcores, _INFO.num_lanes
NW = NC * NS
TPW = N // NW
C = 256
NCHUNK = TPW // C
GPC = C // LANES


def _body(in_hbm, atom_hbm, num_hbm, out_hbm, atom_v, num_v, in_v, out_v):
    wid = lax.axis_index("s") * NC + lax.axis_index("c")

    pltpu.sync_copy(atom_hbm, atom_v)
    pltpu.sync_copy(num_hbm, num_v)

    iota = lax.iota(jnp.int32, LANES)
    mvecs = [(iota + c) & (LANES - 1) for c in range(LANES)]

    def chunk_body(ci, _):
        tb = wid * TPW + ci * C

        pltpu.sync_copy(in_hbm.at[pl.ds(tb * IN_W, C * IN_W)], in_v)

        def group_body(g, _):
            tv = iota + g * LANES
            ibase = tv * IN_W
            obase = tv * OUT_W

            names = plsc.load_gather(in_v, [ibase]).astype(jnp.int32)
            abase = names * AP
            nbases = []
            for j in range(8):
                nid = plsc.load_gather(in_v, [ibase + (33 + j)])
                nbases.append(nid.astype(jnp.int32) * NP)

            K = 8
            for blk in range(DIM // NDIM):
                ab = abase + blk * NDIM
                ob = obase + blk * NDIM
                nb = nbases[blk]
                for c0 in range(0, NDIM, K):
                    avs = [plsc.load_gather(atom_v, [ab + mvecs[c0 + k]])
                           for k in range(K)]
                    nvs = [plsc.load_gather(num_v, [nb + mvecs[c0 + k]])
                           for k in range(K)]
                    for k in range(K):
                        plsc.store_scatter(out_v, [ob + mvecs[c0 + k]],
                                           avs[k] + nvs[k])

            ob = obase + DIM
            for c0 in range(0, NDIM, K):
                vs = [plsc.load_gather(in_v, [ibase + 4 + mvecs[c0 + k]])
                      for k in range(K)]
                for k in range(K):
                    plsc.store_scatter(out_v, [ob + mvecs[c0 + k]], vs[k])
            ob = obase + DIM + NDIM
            svecs = [jnp.where(mv <= 12, mv + 20, mv - 12) for mv in mvecs]
            for c0 in range(0, NDIM, K):
                vs = [plsc.load_gather(in_v, [ibase + svecs[c0 + k]])
                      for k in range(K)]
                for k in range(K):
                    plsc.store_scatter(out_v, [ob + mvecs[c0 + k]], vs[k])
            return _

        lax.fori_loop(0, GPC, group_body, None)

        pltpu.sync_copy(out_v, out_hbm.at[pl.ds(tb * OUT_W, C * OUT_W)])
        return _

    lax.fori_loop(0, NCHUNK, chunk_body, None)


def kernel(inputs, atom_table, num_table):
    mesh = plsc.VectorSubcoreMesh(core_axis_name="c", subcore_axis_name="s")
    run = functools.partial(
        pl.kernel,
        mesh=mesh,
        compiler_params=pltpu.CompilerParams(needs_layout_passes=False),
        out_type=jax.ShapeDtypeStruct((N * OUT_W,), jnp.float32),
        scratch_types=[
            pltpu.VMEM((100 * AP,), jnp.float32),
            pltpu.VMEM((500 * NP,), jnp.float32),
            pltpu.VMEM((C * IN_W,), jnp.float32),
            pltpu.VMEM((C * OUT_W,), jnp.float32),
        ],
    )(_body)
    atom_p = jnp.pad(atom_table, ((0, 0), (0, AP - DIM))).reshape(-1)
    num_p = jnp.pad(num_table, ((0, 0), (0, NP - NDIM))).reshape(-1)
    out = run(inputs.reshape(-1), atom_p, num_p)
    return out.reshape(B, L, OUT_W)


# speedup vs baseline: 11.6545x; 1.0294x over previous
import functools

import jax
import jax.numpy as jnp
from jax import lax
from jax.experimental import pallas as pl
from jax.experimental.pallas import tpu as pltpu
from jax.experimental.pallas import tpu_sc as plsc

B, L = 1024, 200
N = B * L
IN_W = 41
OUT_W = 160
DIM = 128
AP = 128
NDIM = 16
NP = 16

_INFO = plsc.get_sparse_core_info()
NC, NS, LANES = _INFO.num_cores, _INFO.num_subcores, _INFO.num_lanes
NW = NC * NS
TPW = N // NW
C = 256
NCHUNK = TPW // C
GPC = C // LANES


def _body(in_hbm, atom_hbm, num_hbm, out_hbm, atom_v, num_v, in_v, out_v):
    wid = lax.axis_index("s") * NC + lax.axis_index("c")

    pltpu.sync_copy(atom_hbm, atom_v)
    pltpu.sync_copy(num_hbm, num_v)

    iota = lax.iota(jnp.int32, LANES)
    mvecs = [(iota + c) & (LANES - 1) for c in range(LANES)]

    def chunk_body(ci, _):
        tb = wid * TPW + ci * C

        pltpu.sync_copy(in_hbm.at[pl.ds(tb * IN_W, C * IN_W)], in_v)

        def group_body(g, _):
            tv = iota + g * LANES
            ibase = tv * IN_W
            obase = tv * OUT_W

            names = plsc.load_gather(in_v, [ibase]).astype(jnp.int32)
            abase = names * AP
            nbases = []
            for j in range(8):
                nid = plsc.load_gather(in_v, [ibase + (33 + j)])
                nbases.append(nid.astype(jnp.int32) * NP)

            K = 8
            for blk in range(DIM // NDIM):
                ab = abase + blk * NDIM
                ob = obase + blk * NDIM
                nb = nbases[blk]
                for c0 in range(0, NDIM, K):
                    avs = [plsc.load_gather(atom_v, [ab + mvecs[c0 + k]])
                           for k in range(K)]
                    nvs = [plsc.load_gather(num_v, [nb + mvecs[c0 + k]])
                           for k in range(K)]
                    for k in range(K):
                        plsc.store_scatter(out_v, [ob + mvecs[c0 + k]],
                                           avs[k] + nvs[k])

            ob = obase + DIM
            for c0 in range(0, NDIM, K):
                vs = [plsc.load_gather(in_v, [ibase + 4 + mvecs[c0 + k]])
                      for k in range(K)]
                for k in range(K):
                    plsc.store_scatter(out_v, [ob + mvecs[c0 + k]], vs[k])
            ob = obase + DIM + NDIM
            svecs = [jnp.where(mv <= 12, mv + 20, mv - 12) for mv in mvecs]
            for c0 in range(0, NDIM, K):
                vs = [plsc.load_gather(in_v, [ibase + svecs[c0 + k]])
                      for k in range(K)]
                for k in range(K):
                    plsc.store_scatter(out_v, [ob + mvecs[c0 + k]], vs[k])
            return _

        lax.fori_loop(0, GPC, group_body, None)

        pltpu.sync_copy(out_v, out_hbm.at[pl.ds(tb * OUT_W, C * OUT_W)])
        return _

    lax.fori_loop(0, NCHUNK, chunk_body, None)


def kernel(inputs, atom_table, num_table):
    mesh = plsc.VectorSubcoreMesh(core_axis_name="c", subcore_axis_name="s")
    run = functools.partial(
        pl.kernel,
        mesh=mesh,
        compiler_params=pltpu.CompilerParams(needs_layout_passes=False),
        out_type=jax.ShapeDtypeStruct((N * OUT_W,), jnp.float32),
        scratch_types=[
            pltpu.VMEM((100 * AP,), jnp.float32),
            pltpu.VMEM((500 * NP,), jnp.float32),
            pltpu.VMEM((C * IN_W,), jnp.float32),
            pltpu.VMEM((C * OUT_W,), jnp.float32),
        ],
    )(_body)
    atom_p = jnp.pad(atom_table, ((0, 0), (0, AP - DIM))).reshape(-1)
    num_p = jnp.pad(num_table, ((0, 0), (0, NP - NDIM))).reshape(-1)
    out = run(inputs.reshape(-1), atom_p, num_p)
    return out.reshape(B, L, OUT_W)


# speedup vs baseline: 12.9334x; 1.1097x over previous
import functools

import jax
import jax.numpy as jnp
from jax import lax
from jax.experimental import pallas as pl
from jax.experimental.pallas import tpu as pltpu
from jax.experimental.pallas import tpu_sc as plsc

B, L = 1024, 200
N = B * L
IN_W = 41
OUT_W = 160
DIM = 128
NDIM = 16

_INFO = plsc.get_sparse_core_info()
NC, NS, LANES = _INFO.num_cores, _INFO.num_subcores, _INFO.num_lanes
NW = NC * NS
TPW = N // NW
C = 160
NCHUNK = TPW // C
GPC = C // LANES


def _body(in_hbm, atom_hbm, num_hbm, out_hbm, atom_v, num_v, in_v, out_v):
    wid = lax.axis_index("s") * NC + lax.axis_index("c")

    pltpu.sync_copy(atom_hbm, atom_v)
    pltpu.sync_copy(num_hbm, num_v)

    iota = lax.iota(jnp.int32, LANES)
    mvecs = [(iota + c) & (LANES - 1) for c in range(LANES)]
    K = 8

    def chunk_body(ci, _):
        tb = wid * TPW + ci * C

        pltpu.sync_copy(in_hbm.at[pl.ds(tb * IN_W, C * IN_W)], in_v)

        def group_body(g, _):
            tv = iota + g * LANES
            ibase = tv * IN_W

            names = plsc.load_gather(in_v, [ibase]).astype(jnp.int32)
            abase = names * DIM
            nbases = []
            for j in range(8):
                nid = plsc.load_gather(in_v, [ibase + (33 + j)])
                nbases.append(nid.astype(jnp.int32) * NDIM)

            for blk in range(DIM // NDIM):
                ab = abase + blk * NDIM
                nb = nbases[blk]
                for c0 in range(0, NDIM, K):
                    avs = [plsc.load_gather(atom_v, [ab + mvecs[c0 + k]])
                           for k in range(K)]
                    nvs = [plsc.load_gather(num_v, [nb + mvecs[c0 + k]])
                           for k in range(K)]
                    for k in range(K):
                        plsc.store_scatter(
                            out_v, [tv, mvecs[c0 + k] + blk * NDIM],
                            avs[k] + nvs[k])

            for c0 in range(0, NDIM, K):
                vs = [plsc.load_gather(in_v, [ibase + mvecs[c0 + k] + 4])
                      for k in range(K)]
                for k in range(K):
                    plsc.store_scatter(out_v, [tv, mvecs[c0 + k] + DIM], vs[k])
            for c0 in range(0, NDIM, K):
                svs = [jnp.where(mvecs[c0 + k] <= 12, mvecs[c0 + k] + 20,
                                 mvecs[c0 + k] - 12) for k in range(K)]
                vs = [plsc.load_gather(in_v, [ibase + svs[k]]) for k in range(K)]
                for k in range(K):
                    plsc.store_scatter(out_v, [tv, mvecs[c0 + k] + DIM + NDIM],
                                       vs[k])
            return _

        lax.fori_loop(0, GPC, group_body, None)

        pltpu.sync_copy(out_v, out_hbm.at[pl.ds(tb, C), :])
        return _

    lax.fori_loop(0, NCHUNK, chunk_body, None)


def kernel(inputs, atom_table, num_table):
    mesh = plsc.VectorSubcoreMesh(core_axis_name="c", subcore_axis_name="s")
    run = functools.partial(
        pl.kernel,
        mesh=mesh,
        compiler_params=pltpu.CompilerParams(needs_layout_passes=False),
        out_type=jax.ShapeDtypeStruct((N, OUT_W), jnp.float32),
        scratch_types=[
            pltpu.VMEM((100 * DIM,), jnp.float32),
            pltpu.VMEM((500 * NDIM,), jnp.float32),
            pltpu.VMEM((C * IN_W,), jnp.float32),
            pltpu.VMEM((C, OUT_W), jnp.float32),
        ],
    )(_body)
    out = run(inputs.reshape(-1),
              atom_table.reshape(-1),
              num_table.reshape(-1))
    return out.reshape(B, L, OUT_W)


# speedup vs baseline: 18.5436x; 1.4338x over previous
import functools

import jax
import jax.numpy as jnp
from jax import lax
from jax.experimental import pallas as pl
from jax.experimental.pallas import tpu as pltpu
from jax.experimental.pallas import tpu_sc as plsc

B, L = 1024, 200
N = B * L
IN_W = 41
OUT_W = 160
DIM = 128
NDIM = 16

_INFO = plsc.get_sparse_core_info()
NC, NS, LANES = _INFO.num_cores, _INFO.num_subcores, _INFO.num_lanes
NW = NC * NS
TPW = N // NW
C = 80
NCHUNK = TPW // C
GPC = C // LANES
K = 8


def _body(in_hbm, atom_hbm, num_hbm, out_hbm,
          atom_v, num_v, in_v0, in_v1, out_v0, out_v1,
          si0, si1, so0, so1):
    wid = lax.axis_index("s") * NC + lax.axis_index("c")
    tw = wid * TPW

    pltpu.sync_copy(atom_hbm, atom_v)
    pltpu.sync_copy(num_hbm, num_v)

    iota = lax.iota(jnp.int32, LANES)
    mvecs = [(iota + c) & (LANES - 1) for c in range(LANES)]

    def in_copy(ci, iv, sem):
        return pltpu.make_async_copy(
            in_hbm.at[pl.ds(tw + ci * C, C), :], iv, sem)

    def out_copy(ci, ov, sem):
        return pltpu.make_async_copy(
            ov, out_hbm.at[pl.ds(tw + ci * C, C), :], sem)

    def compute(iv, ov):
        def group_body(g, _):
            tv = iota + g * LANES
            zeros = mvecs[0] * 0

            names = plsc.load_gather(iv, [tv, zeros]).astype(jnp.int32)
            abase = names * DIM
            nbases = []
            for j in range(8):
                nid = plsc.load_gather(iv, [tv, zeros + (33 + j)])
                nbases.append(nid.astype(jnp.int32) * NDIM)

            for blk in range(DIM // NDIM):
                ab = abase + blk * NDIM
                nb = nbases[blk]
                for c0 in range(0, NDIM, K):
                    avs = [plsc.load_gather(atom_v, [ab + mvecs[c0 + k]])
                           for k in range(K)]
                    nvs = [plsc.load_gather(num_v, [nb + mvecs[c0 + k]])
                           for k in range(K)]
                    for k in range(K):
                        plsc.store_scatter(
                            ov, [tv, mvecs[c0 + k] + blk * NDIM],
                            avs[k] + nvs[k])

            for c0 in range(0, NDIM, K):
                vs = [plsc.load_gather(iv, [tv, mvecs[c0 + k] + 4])
                      for k in range(K)]
                for k in range(K):
                    plsc.store_scatter(ov, [tv, mvecs[c0 + k] + DIM], vs[k])
            for c0 in range(0, NDIM, K):
                svs = [jnp.where(mvecs[c0 + k] <= 12, mvecs[c0 + k] + 20,
                                 mvecs[c0 + k] - 12) for k in range(K)]
                vs = [plsc.load_gather(iv, [tv, svs[k]]) for k in range(K)]
                for k in range(K):
                    plsc.store_scatter(ov, [tv, mvecs[c0 + k] + DIM + NDIM],
                                       vs[k])
            return _

        lax.fori_loop(0, GPC, group_body, None)

    bufs = ((in_v0, out_v0, si0, so0), (in_v1, out_v1, si1, so1))

    in_copy(0, in_v0, si0).start()
    in_copy(1, in_v1, si1).start()

    def super_body(i, _):
        for b, (iv, ov, sin, son) in enumerate(bufs):
            ci = 2 * i + b
            in_copy(ci, iv, sin).wait()

            @pl.when(i > 0)
            def _drain():
                out_copy(ci - 2, ov, son).wait()

            compute(iv, ov)
            out_copy(ci, ov, son).start()

            @pl.when(ci + 2 < NCHUNK)
            def _prefetch():
                in_copy(ci + 2, iv, sin).start()
        return _

    lax.fori_loop(0, NCHUNK // 2, super_body, None)

    out_copy(NCHUNK - 2, out_v0, so0).wait()
    out_copy(NCHUNK - 1, out_v1, so1).wait()


def kernel(inputs, atom_table, num_table):
    mesh = plsc.VectorSubcoreMesh(core_axis_name="c", subcore_axis_name="s")
    run = functools.partial(
        pl.kernel,
        mesh=mesh,
        compiler_params=pltpu.CompilerParams(needs_layout_passes=False),
        out_type=jax.ShapeDtypeStruct((N, OUT_W), jnp.float32),
        scratch_types=[
            pltpu.VMEM((100 * DIM,), jnp.float32),
            pltpu.VMEM((500 * NDIM,), jnp.float32),
            pltpu.VMEM((C, IN_W), jnp.float32),
            pltpu.VMEM((C, IN_W), jnp.float32),
            pltpu.VMEM((C, OUT_W), jnp.float32),
            pltpu.VMEM((C, OUT_W), jnp.float32),
            pltpu.SemaphoreType.DMA,
            pltpu.SemaphoreType.DMA,
            pltpu.SemaphoreType.DMA,
            pltpu.SemaphoreType.DMA,
        ],
    )(_body)
    out = run(inputs.reshape(N, IN_W),
              atom_table.reshape(-1),
              num_table.reshape(-1))
    return out.reshape(B, L, OUT_W)
